# Initial kernel scaffold; baseline (speedup 1.0000x reference)
import functools

import jax
import jax.numpy as jnp
from jax import lax
from jax.experimental import pallas as pl
from jax.experimental.pallas import tpu as pltpu
from jax.experimental.pallas import tpu_sc as plsc

L = 50
D = 64
HID = 128
V = 100000
NW = 32
IDX_COLS = 100
ROWS_PER_W = 256
VT = 12500
NT = V // VT


def _pool_kernel(idx_hbm, emb_hbm, out_hbm, idx_v, rows_v, acc_v, sem):
    wid = lax.axis_index("s") * 2 + lax.axis_index("c")

    pltpu.sync_copy(idx_hbm.at[pl.ds(wid * ROWS_PER_W, ROWS_PER_W)], idx_v)

    zero = jnp.zeros((16,), jnp.float32)
    for l_ in range(L):
        for c_ in range(D // 16):
            acc_v[l_, pl.ds(c_ * 16, 16)] = zero

    def body(j, carry):
        pltpu.async_copy(emb_hbm.at[idx_v.at[j]], rows_v, sem).wait()
        for l_ in range(L):
            for c_ in range(D // 16):
                sl = pl.ds(c_ * 16, 16)
                plsc.addupdate(acc_v.at[l_, sl], rows_v[l_, sl] + rows_v[L + l_, sl])
        return carry

    lax.fori_loop(0, ROWS_PER_W, body, 0)
    pltpu.sync_copy(acc_v, out_hbm.at[wid])


def _pool(idx2, emb):
    mesh = plsc.VectorSubcoreMesh(core_axis_name="c", subcore_axis_name="s")
    return pl.kernel(
        _pool_kernel,
        out_type=jax.ShapeDtypeStruct((NW, L, D), jnp.float32),
        mesh=mesh,
        scratch_types=[
            pltpu.VMEM((ROWS_PER_W, IDX_COLS), jnp.int32),
            pltpu.VMEM((IDX_COLS, D), jnp.float32),
            pltpu.VMEM((L, D), jnp.float32),
            pltpu.SemaphoreType.DMA,
        ],
    )(idx2, emb)


def _head_body(parts_ref, w1t_ref, b1_ref, h_ref):
    pooled = jnp.sum(parts_ref[...], axis=0)
    h = jnp.dot(pooled, w1t_ref[...], preferred_element_type=jnp.float32)
    h_ref[...] = jnp.maximum(h + b1_ref[...], 0.0)


def _logits_body(h_ref, w2_ref, b2_ref, out_ref, m_out, s_out, m_ref, s_ref):
    t = pl.program_id(0)
    logits = lax.dot_general(
        h_ref[...], w2_ref[...], (((1,), (1,)), ((), ())),
        preferred_element_type=jnp.float32) + b2_ref[...]
    out_ref[...] = logits
    m_tile = jnp.max(logits, axis=1, keepdims=True)

    @pl.when(t == 0)
    def _():
        m_ref[...] = m_tile
        s_ref[...] = jnp.sum(jnp.exp(logits - m_tile), axis=1, keepdims=True)

    @pl.when(t > 0)
    def _():
        m_prev = m_ref[...]
        m_new = jnp.maximum(m_prev, m_tile)
        m_ref[...] = m_new
        s_ref[...] = (s_ref[...] * jnp.exp(m_prev - m_new)
                      + jnp.sum(jnp.exp(logits - m_new), axis=1, keepdims=True))

    @pl.when(t == NT - 1)
    def _():
        m_out[...] = m_ref[...]
        s_out[...] = s_ref[...]


def _sub_body(lg_ref, m_ref, s_ref, out_ref):
    lse = m_ref[...] + jnp.log(s_ref[...])
    out_ref[...] = lg_ref[...] - lse


def kernel(inputs, emb, W1, b1, W2, b2):
    idx2 = inputs.astype(jnp.int32).reshape(ROWS_PER_W * NW, IDX_COLS)
    parts = _pool(idx2, emb)

    h = pl.pallas_call(
        _head_body,
        out_shape=jax.ShapeDtypeStruct((L, HID), jnp.float32),
    )(parts, W1.T, b1.reshape(1, HID))

    logits, m, s = pl.pallas_call(
        _logits_body,
        grid=(NT,),
        in_specs=[
            pl.BlockSpec((L, HID), lambda t: (0, 0)),
            pl.BlockSpec((VT, HID), lambda t: (t, 0)),
            pl.BlockSpec((1, VT), lambda t: (0, t)),
        ],
        out_specs=[
            pl.BlockSpec((L, VT), lambda t: (0, t)),
            pl.BlockSpec((L, 1), lambda t: (0, 0)),
            pl.BlockSpec((L, 1), lambda t: (0, 0)),
        ],
        out_shape=[
            jax.ShapeDtypeStruct((L, V), jnp.float32),
            jax.ShapeDtypeStruct((L, 1), jnp.float32),
            jax.ShapeDtypeStruct((L, 1), jnp.float32),
        ],
        scratch_shapes=[
            pltpu.VMEM((L, 1), jnp.float32),
            pltpu.VMEM((L, 1), jnp.float32),
        ],
    )(h, W2, b2.reshape(1, V))

    out = pl.pallas_call(
        _sub_body,
        grid=(NT,),
        in_specs=[
            pl.BlockSpec((L, VT), lambda t: (0, t)),
            pl.BlockSpec((L, 1), lambda t: (0, 0)),
            pl.BlockSpec((L, 1), lambda t: (0, 0)),
        ],
        out_specs=pl.BlockSpec((L, VT), lambda t: (0, t)),
        out_shape=jax.ShapeDtypeStruct((L, V), jnp.float32),
        input_output_aliases={0: 0},
    )(logits, m, s)
    return out


# speedup vs baseline: 6.3206x; 6.3206x over previous
import functools

import jax
import jax.numpy as jnp
from jax import lax
from jax.experimental import pallas as pl
from jax.experimental.pallas import tpu as pltpu
from jax.experimental.pallas import tpu_sc as plsc

L = 50
D = 64
HID = 128
V = 100000
NW = 32
IDX_COLS = 100
ROWS_PER_W = 256
VT = 12500
NT = V // VT


def _pool_kernel(idx_hbm, emb_hbm, out_hbm, idx_v, rows_v, acc_v, sem):
    wid = lax.axis_index("s") * 2 + lax.axis_index("c")

    pltpu.sync_copy(idx_hbm.at[pl.ds(wid * ROWS_PER_W, ROWS_PER_W)], idx_v)

    zero = jnp.zeros((16,), jnp.float32)
    for l_ in range(L):
        for c_ in range(D // 16):
            acc_v[l_, pl.ds(c_ * 16, 16)] = zero

    def body(j, carry):
        pltpu.async_copy(emb_hbm.at[idx_v.at[j]], rows_v, sem).wait()
        for l_ in range(L):
            for c_ in range(D // 16):
                sl = pl.ds(c_ * 16, 16)
                plsc.addupdate(acc_v.at[l_, sl], rows_v[l_, sl] + rows_v[L + l_, sl])
        return carry

    lax.fori_loop(0, ROWS_PER_W, body, 0)
    pltpu.sync_copy(acc_v, out_hbm.at[wid])


def _pool(idx2, emb):
    mesh = plsc.VectorSubcoreMesh(core_axis_name="c", subcore_axis_name="s")
    return pl.kernel(
        _pool_kernel,
        out_type=jax.ShapeDtypeStruct((NW, L, D), jnp.float32),
        mesh=mesh,
        scratch_types=[
            pltpu.VMEM((ROWS_PER_W, IDX_COLS), jnp.int32),
            pltpu.VMEM((IDX_COLS, D), jnp.float32),
            pltpu.VMEM((L, D), jnp.float32),
            pltpu.SemaphoreType.DMA,
        ],
        compiler_params=pltpu.CompilerParams(use_tc_tiling_on_sc=False),
    )(idx2, emb)


def _head_body(parts_ref, w1t_ref, b1_ref, h_ref):
    pooled = jnp.sum(parts_ref[...], axis=0)
    h = jnp.dot(pooled, w1t_ref[...], preferred_element_type=jnp.float32)
    h_ref[...] = jnp.maximum(h + b1_ref[...], 0.0)


def _logits_body(h_ref, w2_ref, b2_ref, out_ref, m_out, s_out, m_ref, s_ref):
    t = pl.program_id(0)
    logits = lax.dot_general(
        h_ref[...], w2_ref[0], (((1,), (1,)), ((), ())),
        preferred_element_type=jnp.float32) + b2_ref[0]
    out_ref[0] = logits
    m_tile = jnp.max(logits, axis=1, keepdims=True)

    @pl.when(t == 0)
    def _():
        m_ref[...] = m_tile
        s_ref[...] = jnp.sum(jnp.exp(logits - m_tile), axis=1, keepdims=True)

    @pl.when(t > 0)
    def _():
        m_prev = m_ref[...]
        m_new = jnp.maximum(m_prev, m_tile)
        m_ref[...] = m_new
        s_ref[...] = (s_ref[...] * jnp.exp(m_prev - m_new)
                      + jnp.sum(jnp.exp(logits - m_new), axis=1, keepdims=True))

    @pl.when(t == NT - 1)
    def _():
        m_out[...] = m_ref[...]
        s_out[...] = s_ref[...]


def _sub_body(lg_ref, m_ref, s_ref, out_ref):
    lse = m_ref[...] + jnp.log(s_ref[...])
    out_ref[0] = lg_ref[0] - lse


def kernel(inputs, emb, W1, b1, W2, b2):
    idx2 = inputs.astype(jnp.int32).reshape(ROWS_PER_W * NW, IDX_COLS)
    parts = _pool(idx2, emb)

    h = pl.pallas_call(
        _head_body,
        out_shape=jax.ShapeDtypeStruct((L, HID), jnp.float32),
    )(parts, W1.T, b1.reshape(1, HID))

    logits3, m, s = pl.pallas_call(
        _logits_body,
        grid=(NT,),
        in_specs=[
            pl.BlockSpec((L, HID), lambda t: (0, 0)),
            pl.BlockSpec((1, VT, HID), lambda t: (t, 0, 0)),
            pl.BlockSpec((1, 1, VT), lambda t: (t, 0, 0)),
        ],
        out_specs=[
            pl.BlockSpec((1, L, VT), lambda t: (t, 0, 0)),
            pl.BlockSpec((L, 1), lambda t: (0, 0)),
            pl.BlockSpec((L, 1), lambda t: (0, 0)),
        ],
        out_shape=[
            jax.ShapeDtypeStruct((NT, L, VT), jnp.float32),
            jax.ShapeDtypeStruct((L, 1), jnp.float32),
            jax.ShapeDtypeStruct((L, 1), jnp.float32),
        ],
        scratch_shapes=[
            pltpu.VMEM((L, 1), jnp.float32),
            pltpu.VMEM((L, 1), jnp.float32),
        ],
    )(h, W2.reshape(NT, VT, HID), b2.reshape(NT, 1, VT))

    out3 = pl.pallas_call(
        _sub_body,
        grid=(NT,),
        in_specs=[
            pl.BlockSpec((1, L, VT), lambda t: (t, 0, 0)),
            pl.BlockSpec((L, 1), lambda t: (0, 0)),
            pl.BlockSpec((L, 1), lambda t: (0, 0)),
        ],
        out_specs=pl.BlockSpec((1, L, VT), lambda t: (t, 0, 0)),
        out_shape=jax.ShapeDtypeStruct((NT, L, VT), jnp.float32),
        input_output_aliases={0: 0},
    )(logits3, m, s)
    return out3.swapaxes(0, 1).reshape(L, V)


# speedup vs baseline: 12.5108x; 1.9794x over previous
import functools

import jax
import jax.numpy as jnp
from jax import lax
from jax.experimental import pallas as pl
from jax.experimental.pallas import tpu as pltpu
from jax.experimental.pallas import tpu_sc as plsc

L = 50
D = 64
HID = 128
V = 100000
NW = 32
VT = 12500
NT = V // VT


IDXC = 128
NBATCH = (16384 * L) // IDXC
BPW = NBATCH // NW
NBUF = 4


def _pool_kernel(idx_hbm, emb_hbm, out_hbm, idx_v, rows_v, acc_v, s0, s1, s2, s3):
    sems = (s0, s1, s2, s3)
    wid = lax.axis_index("s") * 2 + lax.axis_index("c")
    base = wid * BPW

    pltpu.sync_copy(idx_hbm.at[pl.ds(base, BPW)], idx_v)

    zero = jnp.zeros((16,), jnp.float32)
    for l_ in range(L):
        for c_ in range(D // 16):
            acc_v[l_, pl.ds(c_ * 16, 16)] = zero

    def issue(j, b):
        pltpu.async_copy(emb_hbm.at[idx_v.at[j]], rows_v.at[b], sems[b])

    for b in range(NBUF):
        issue(b, b)

    def body(g, carry):
        for b in range(NBUF):
            j = g * NBUF + b
            pltpu.make_async_copy(
                emb_hbm.at[idx_v.at[j]], rows_v.at[b], sems[b]).wait()
            l = lax.shift_right_logical(base + j, 7)
            for c_ in range(D // 16):
                sl = pl.ds(c_ * 16, 16)
                sa = rows_v[b, 0, sl]
                sb = rows_v[b, 1, sl]
                for r_ in range(2, IDXC, 2):
                    sa = sa + rows_v[b, r_, sl]
                    sb = sb + rows_v[b, r_ + 1, sl]
                plsc.addupdate(acc_v.at[l, sl], sa + sb)
            issue(jnp.minimum(j + NBUF, BPW - 1), b)
        return carry

    lax.fori_loop(0, BPW // NBUF, body, 0)
    for b in range(NBUF):
        pltpu.make_async_copy(
            emb_hbm.at[idx_v.at[0]], rows_v.at[b], sems[b]).wait()
    pltpu.sync_copy(acc_v, out_hbm.at[wid])


def _pool(idxT, emb):
    mesh = plsc.VectorSubcoreMesh(core_axis_name="c", subcore_axis_name="s")
    return pl.kernel(
        _pool_kernel,
        out_type=jax.ShapeDtypeStruct((NW, L, D), jnp.float32),
        mesh=mesh,
        scratch_types=[
            pltpu.VMEM((BPW, IDXC), jnp.int32),
            pltpu.VMEM((NBUF, IDXC, D), jnp.float32),
            pltpu.VMEM((L, D), jnp.float32),
            pltpu.SemaphoreType.DMA,
            pltpu.SemaphoreType.DMA,
            pltpu.SemaphoreType.DMA,
            pltpu.SemaphoreType.DMA,
        ],
        compiler_params=pltpu.CompilerParams(use_tc_tiling_on_sc=False),
    )(idxT, emb)


def _head_body(parts_ref, w1t_ref, b1_ref, h_ref):
    pooled = jnp.sum(parts_ref[...], axis=0)
    h = jnp.dot(pooled, w1t_ref[...], preferred_element_type=jnp.float32)
    h_ref[...] = jnp.maximum(h + b1_ref[...], 0.0)


def _logits_body(h_ref, w2_ref, b2_ref, out_ref, m_out, s_out, m_ref, s_ref):
    t = pl.program_id(0)
    logits = lax.dot_general(
        h_ref[...], w2_ref[0], (((1,), (1,)), ((), ())),
        preferred_element_type=jnp.float32) + b2_ref[0]
    out_ref[0] = logits
    m_tile = jnp.max(logits, axis=1, keepdims=True)

    @pl.when(t == 0)
    def _():
        m_ref[...] = m_tile
        s_ref[...] = jnp.sum(jnp.exp(logits - m_tile), axis=1, keepdims=True)

    @pl.when(t > 0)
    def _():
        m_prev = m_ref[...]
        m_new = jnp.maximum(m_prev, m_tile)
        m_ref[...] = m_new
        s_ref[...] = (s_ref[...] * jnp.exp(m_prev - m_new)
                      + jnp.sum(jnp.exp(logits - m_new), axis=1, keepdims=True))

    @pl.when(t == NT - 1)
    def _():
        m_out[...] = m_ref[...]
        s_out[...] = s_ref[...]


def _sub_body(lg_ref, m_ref, s_ref, out_ref):
    lse = m_ref[...] + jnp.log(s_ref[...])
    out_ref[0] = lg_ref[0] - lse


def kernel(inputs, emb, W1, b1, W2, b2):
    idxT = inputs.astype(jnp.int32).T.reshape(NBATCH, IDXC)
    parts = _pool(idxT, emb)

    h = pl.pallas_call(
        _head_body,
        out_shape=jax.ShapeDtypeStruct((L, HID), jnp.float32),
    )(parts, W1.T, b1.reshape(1, HID))

    logits3, m, s = pl.pallas_call(
        _logits_body,
        grid=(NT,),
        in_specs=[
            pl.BlockSpec((L, HID), lambda t: (0, 0)),
            pl.BlockSpec((1, VT, HID), lambda t: (t, 0, 0)),
            pl.BlockSpec((1, 1, VT), lambda t: (t, 0, 0)),
        ],
        out_specs=[
            pl.BlockSpec((1, L, VT), lambda t: (t, 0, 0)),
            pl.BlockSpec((L, 1), lambda t: (0, 0)),
            pl.BlockSpec((L, 1), lambda t: (0, 0)),
        ],
        out_shape=[
            jax.ShapeDtypeStruct((NT, L, VT), jnp.float32),
            jax.ShapeDtypeStruct((L, 1), jnp.float32),
            jax.ShapeDtypeStruct((L, 1), jnp.float32),
        ],
        scratch_shapes=[
            pltpu.VMEM((L, 1), jnp.float32),
            pltpu.VMEM((L, 1), jnp.float32),
        ],
    )(h, W2.reshape(NT, VT, HID), b2.reshape(NT, 1, VT))

    out3 = pl.pallas_call(
        _sub_body,
        grid=(NT,),
        in_specs=[
            pl.BlockSpec((1, L, VT), lambda t: (t, 0, 0)),
            pl.BlockSpec((L, 1), lambda t: (0, 0)),
            pl.BlockSpec((L, 1), lambda t: (0, 0)),
        ],
        out_specs=pl.BlockSpec((1, L, VT), lambda t: (t, 0, 0)),
        out_shape=jax.ShapeDtypeStruct((NT, L, VT), jnp.float32),
        input_output_aliases={0: 0},
    )(logits3, m, s)
    return out3.swapaxes(0, 1).reshape(L, V)


# speedup vs baseline: 12.6054x; 1.0076x over previous
import functools

import jax
import jax.numpy as jnp
from jax import lax
from jax.experimental import pallas as pl
from jax.experimental.pallas import tpu as pltpu
from jax.experimental.pallas import tpu_sc as plsc

L = 50
D = 64
INPUT_V = 100000
HID = 128
V = 100000
NW = 32
VT = 12500
NT = V // VT


IDXC = 128
NBATCH = (16384 * L) // IDXC
BPW = NBATCH // NW
NBUF = 4


def _pool_kernel(idx_hbm, emb_hbm, out_hbm, idx_v, rows_v, acc_v, s0, s1, s2, s3):
    sems = (s0, s1, s2, s3)
    wid = lax.axis_index("s") * 2 + lax.axis_index("c")
    base = wid * BPW

    pltpu.sync_copy(idx_hbm.at[pl.ds(base, BPW)], idx_v)

    zero = jnp.zeros((16,), jnp.float32)
    for l_ in range(L):
        for c_ in range(D // 16):
            acc_v[l_, pl.ds(c_ * 16, 16)] = zero

    def issue(j, b):
        pltpu.async_copy(emb_hbm.at[idx_v.at[j]], rows_v.at[b], sems[b])

    for b in range(NBUF):
        issue(b, b)

    def body(g, carry):
        for b in range(NBUF):
            j = g * NBUF + b
            pltpu.make_async_copy(
                emb_hbm.at[idx_v.at[j]], rows_v.at[b], sems[b]).wait()
            l = lax.shift_right_logical(base + j, 7)
            for c_ in range(D // 16):
                sl = pl.ds(c_ * 16, 16)
                sa = rows_v[b, 0, sl]
                sb = rows_v[b, 1, sl]
                for r_ in range(2, IDXC, 2):
                    sa = sa + rows_v[b, r_, sl]
                    sb = sb + rows_v[b, r_ + 1, sl]
                plsc.addupdate(acc_v.at[l, sl], sa + sb)
            issue(jnp.minimum(j + NBUF, BPW - 1), b)
        return carry

    lax.fori_loop(0, BPW // NBUF, body, 0)
    for b in range(NBUF):
        pltpu.make_async_copy(
            emb_hbm.at[idx_v.at[0]], rows_v.at[b], sems[b]).wait()
    pltpu.sync_copy(acc_v, out_hbm.at[wid])


def _pool(idxT, emb):
    mesh = plsc.VectorSubcoreMesh(core_axis_name="c", subcore_axis_name="s")
    return pl.kernel(
        _pool_kernel,
        out_type=jax.ShapeDtypeStruct((NW, L, D), jnp.float32),
        mesh=mesh,
        scratch_types=[
            pltpu.VMEM((BPW, IDXC), jnp.int32),
            pltpu.VMEM((NBUF, IDXC, D), jnp.float32),
            pltpu.VMEM((L, D), jnp.float32),
            pltpu.SemaphoreType.DMA,
            pltpu.SemaphoreType.DMA,
            pltpu.SemaphoreType.DMA,
            pltpu.SemaphoreType.DMA,
        ],
        compiler_params=pltpu.CompilerParams(use_tc_tiling_on_sc=False),
    )(idxT, emb)


def _mlp_body(parts_ref, w1_ref, b1_ref, w2_ref, b2_ref, out_ref, lse_ref,
              h_ref, m_ref, s_ref):
    t = pl.program_id(0)

    @pl.when(t == 0)
    def _():
        pooled = jnp.sum(parts_ref[...], axis=0)
        h = lax.dot_general(pooled, w1_ref[...], (((1,), (1,)), ((), ())),
                            preferred_element_type=jnp.float32)
        h_ref[...] = jnp.maximum(h + b1_ref[...], 0.0)

    logits = lax.dot_general(
        h_ref[...], w2_ref[0], (((1,), (1,)), ((), ())),
        preferred_element_type=jnp.float32) + b2_ref[0]
    out_ref[0] = logits
    m_tile = jnp.max(logits, axis=1, keepdims=True)

    @pl.when(t == 0)
    def _():
        m_ref[...] = m_tile
        s_ref[...] = jnp.sum(jnp.exp(logits - m_tile), axis=1, keepdims=True)

    @pl.when(t > 0)
    def _():
        m_prev = m_ref[...]
        m_new = jnp.maximum(m_prev, m_tile)
        m_ref[...] = m_new
        s_ref[...] = (s_ref[...] * jnp.exp(m_prev - m_new)
                      + jnp.sum(jnp.exp(logits - m_new), axis=1, keepdims=True))

    @pl.when(t == NT - 1)
    def _():
        lse_ref[...] = m_ref[...] + jnp.log(s_ref[...])


def kernel(inputs, emb, W1, b1, W2, b2):
    idxT = inputs.astype(jnp.int32).T.reshape(NBATCH, IDXC)
    parts = _pool(idxT, emb)

    logits3, lse = pl.pallas_call(
        _mlp_body,
        grid=(NT,),
        in_specs=[
            pl.BlockSpec((NW, L, D), lambda t: (0, 0, 0)),
            pl.BlockSpec((HID, D), lambda t: (0, 0)),
            pl.BlockSpec((1, HID), lambda t: (0, 0)),
            pl.BlockSpec((1, VT, HID), lambda t: (t, 0, 0)),
            pl.BlockSpec((1, 1, VT), lambda t: (t, 0, 0)),
        ],
        out_specs=[
            pl.BlockSpec((1, L, VT), lambda t: (t, 0, 0)),
            pl.BlockSpec((L, 1), lambda t: (0, 0)),
        ],
        out_shape=[
            jax.ShapeDtypeStruct((NT, L, VT), jnp.float32),
            jax.ShapeDtypeStruct((L, 1), jnp.float32),
        ],
        scratch_shapes=[
            pltpu.VMEM((L, HID), jnp.float32),
            pltpu.VMEM((L, 1), jnp.float32),
            pltpu.VMEM((L, 1), jnp.float32),
        ],
    )(parts, W1, b1.reshape(1, HID), W2.reshape(NT, VT, HID),
      b2.reshape(NT, 1, VT))

    return (logits3 - lse[None]).swapaxes(0, 1).reshape(L, V)
